# baseline (speedup 1.0000x reference)
import functools

import jax
import jax.numpy as jnp
from jax.experimental import pallas as pl
from jax.experimental.pallas import tpu as pltpu


def _vmem_gather_kernel(a_ref, s_ref, table_ref, out_ref,
                        *, tile_b: int, sub_rows: int, win: int):
    base = pl.program_id(0) * tile_b
    for r in range(tile_b):
        a = pl.multiple_of(a_ref[base + r], 8)
        chunk = table_ref[pl.ds(a, win), :]
        rot = pltpu.roll(chunk, s_ref[base + r], axis=0)
        d = (r * sub_rows) & 7
        out_ref[pl.ds(r * sub_rows, sub_rows), :] = rot[d:d + sub_rows, :]


def kernel(labels, table, force_drop_ids):
    (B,) = labels.shape
    V, H = table.shape
    cfg_row = V - 1
    sub_rows = H // 128
    win = ((sub_rows + 14) // 8) * 8
    n_rows = V * sub_rows
    rows_pad = ((n_rows + 7) // 8) * 8

    labels = labels.astype(jnp.int32)
    force_drop_ids = force_drop_ids.astype(jnp.int32)

    tile_b = 256
    while B % tile_b != 0:
        tile_b //= 2
    n_b = B // tile_b

    row = jnp.where(force_drop_ids == 1, cfg_row, labels)
    row = jnp.clip(row, 0, cfg_row)
    slab = row * sub_rows
    a = jnp.minimum((slab >> 3) << 3, rows_pad - win)
    d = (jnp.arange(B, dtype=jnp.int32) * sub_rows) & 7
    shift = (d - (slab - a)) & (win - 1)

    table2 = table.reshape(n_rows, 128)
    itemsize = jnp.dtype(table.dtype).itemsize

    grid_spec = pltpu.PrefetchScalarGridSpec(
        num_scalar_prefetch=2,
        grid=(n_b,),
        in_specs=[
            pl.BlockSpec((n_rows, 128), lambda i, av, sv: (0, 0),
                         pipeline_mode=pl.Buffered(1)),
        ],
        out_specs=pl.BlockSpec((tile_b * sub_rows, 128),
                               lambda i, av, sv: (i, 0)),
    )
    out = pl.pallas_call(
        functools.partial(_vmem_gather_kernel, tile_b=tile_b,
                          sub_rows=sub_rows, win=win),
        out_shape=jax.ShapeDtypeStruct((B * sub_rows, 128), table.dtype),
        grid_spec=grid_spec,
        compiler_params=pltpu.CompilerParams(
            dimension_semantics=("arbitrary",),
            vmem_limit_bytes=100 * 1024 * 1024,
            disable_bounds_checks=True,
        ),
        cost_estimate=pl.CostEstimate(
            flops=0,
            transcendentals=0,
            bytes_accessed=(V * H + B * H) * itemsize + 8 * B),
    )(a, shift, table2)
    return out.reshape(B, H)


# speedup vs baseline: 3.0156x; 3.0156x over previous
import functools

import jax
import jax.numpy as jnp
from jax.experimental import pallas as pl
from jax.experimental.pallas import tpu as pltpu


def _vmem_gather_kernel(a_ref, s_ref, table_ref, out_ref,
                        *, tile_b: int):
    base = pl.program_id(0) * tile_b
    for r in range(tile_b):
        a = pl.multiple_of(a_ref[base + r], 8)
        chunk = table_ref[pl.ds(a, 8), :]
        rot = pltpu.roll(chunk, s_ref[base + r], axis=0)
        d = r & 7
        out_ref[pl.ds(r, 1), :] = rot[d:d + 1, :]


def kernel(labels, table, force_drop_ids):
    (B,) = labels.shape
    V, H = table.shape
    cfg_row = V - 1

    labels = labels.astype(jnp.int32)
    force_drop_ids = force_drop_ids.astype(jnp.int32)

    tile_b = 256
    while B % tile_b != 0:
        tile_b //= 2
    n_b = B // tile_b

    row = jnp.where(force_drop_ids == 1, cfg_row, labels)
    row = jnp.clip(row, 0, cfg_row)
    a = (row >> 3) << 3
    d = jnp.arange(B, dtype=jnp.int32) & 7
    shift = (d - (row - a)) & 7

    itemsize = jnp.dtype(table.dtype).itemsize

    grid_spec = pltpu.PrefetchScalarGridSpec(
        num_scalar_prefetch=2,
        grid=(n_b,),
        in_specs=[
            pl.BlockSpec((V, H), lambda i, av, sv: (0, 0),
                         pipeline_mode=pl.Buffered(1)),
        ],
        out_specs=pl.BlockSpec((tile_b, H), lambda i, av, sv: (i, 0)),
    )
    out = pl.pallas_call(
        functools.partial(_vmem_gather_kernel, tile_b=tile_b),
        out_shape=jax.ShapeDtypeStruct((B, H), table.dtype),
        grid_spec=grid_spec,
        compiler_params=pltpu.CompilerParams(
            dimension_semantics=("arbitrary",),
            vmem_limit_bytes=100 * 1024 * 1024,
            disable_bounds_checks=True,
        ),
        cost_estimate=pl.CostEstimate(
            flops=0,
            transcendentals=0,
            bytes_accessed=(V * H + B * H) * itemsize + 8 * B),
    )(a, shift, table)
    return out


# speedup vs baseline: 3.0293x; 1.0045x over previous
import functools

import jax
import jax.numpy as jnp
from jax.experimental import pallas as pl
from jax.experimental.pallas import tpu as pltpu


def _vmem_gather_kernel(a_ref, s_ref, table_ref, out_ref,
                        *, tile_b: int):
    base = pl.program_id(0) * tile_b
    for r in range(tile_b):
        a = pl.multiple_of(a_ref[base + r], 8)
        chunk = table_ref[pl.ds(a, 8), :]
        rot = pltpu.roll(chunk, s_ref[base + r], axis=0)
        d = r & 7
        out_ref[pl.ds(r, 1), :] = rot[d:d + 1, :]


def kernel(labels, table, force_drop_ids):
    (B,) = labels.shape
    V, H = table.shape
    cfg_row = V - 1

    labels = labels.astype(jnp.int32)
    force_drop_ids = force_drop_ids.astype(jnp.int32)

    tile_b = 256
    while B % tile_b != 0:
        tile_b //= 2
    n_b = B // tile_b

    row = jnp.where(force_drop_ids == 1, cfg_row, labels)
    row = jnp.clip(row, 0, cfg_row)
    a = (row >> 3) << 3
    d = jnp.arange(B, dtype=jnp.int32) & 7
    shift = (d - (row - a)) & 7

    itemsize = jnp.dtype(table.dtype).itemsize

    grid_spec = pltpu.PrefetchScalarGridSpec(
        num_scalar_prefetch=2,
        grid=(n_b,),
        in_specs=[
            pl.BlockSpec((V, H), lambda i, av, sv: (0, 0),
                         pipeline_mode=pl.Buffered(1)),
        ],
        out_specs=pl.BlockSpec((tile_b, H), lambda i, av, sv: (i, 0)),
    )
    out = pl.pallas_call(
        functools.partial(_vmem_gather_kernel, tile_b=tile_b),
        out_shape=jax.ShapeDtypeStruct((B, H), table.dtype),
        grid_spec=grid_spec,
        compiler_params=pltpu.CompilerParams(
            dimension_semantics=("parallel",),
            vmem_limit_bytes=100 * 1024 * 1024,
            disable_bounds_checks=True,
        ),
        cost_estimate=pl.CostEstimate(
            flops=0,
            transcendentals=0,
            bytes_accessed=(V * H + B * H) * itemsize + 8 * B),
    )(a, shift, table)
    return out
